# Initial kernel scaffold; baseline (speedup 1.0000x reference)
import jax
import jax.numpy as jnp
from jax.experimental import pallas as pl


def kernel(thresholds, rules_left, rules_right, binary_ops, left, right):
    raise NotImplementedError("write your pallas kernel here")


# speedup vs baseline: 693.4900x; 693.4900x over previous
import functools

import jax
import jax.numpy as jnp
from jax import lax
from jax.experimental import pallas as pl
from jax.experimental.pallas import tpu as pltpu
from jax.experimental.pallas import tpu_sc as plsc

_N = 10000
_L = 16
_G = _N // _L
_BITS = 14


def _winners(idx):
    ids1 = jnp.arange(1, _N + 1, dtype=jnp.int32)
    w = jnp.zeros(_N, jnp.int32)
    for b in range(_BITS):
        plane = jnp.zeros(_N, jnp.bool_).at[idx].set((ids1 >> b) & 1 != 0)
        w = w | (jnp.where(plane, jnp.int32(1), jnp.int32(0)) << b)
    self_ids = jnp.arange(_N, dtype=jnp.int32)
    return jnp.where(w > 0, w - 1, self_ids)


def _build():
    mesh = plsc.VectorSubcoreMesh(core_axis_name="c", subcore_axis_name="s")

    @functools.partial(
        pl.kernel,
        mesh=mesh,
        out_type=jax.ShapeDtypeStruct((_N,), jnp.int32),
        compiler_params=pltpu.CompilerParams(needs_layout_passes=False),
        scratch_types=[
            pltpu.VMEM((_N,), jnp.int32),
            pltpu.VMEM((_N,), jnp.int32),
            pltpu.VMEM((_N,), jnp.int32),
        ],
    )
    def k(wl_hbm, wr_hbm, out_hbm, wl_v, wr_v, mask_v):
        cid = lax.axis_index("c")
        sid = lax.axis_index("s")

        @pl.when((cid == 0) & (sid == 0))
        def _():
            pltpu.sync_copy(wl_hbm, wl_v)
            pltpu.sync_copy(wr_hbm, wr_v)
            lanes = lax.iota(jnp.int32, _L)

            def init_body(g, c):
                mask_v[pl.ds(g * _L, _L)] = jnp.zeros(_L, jnp.int32)
                return c

            lax.fori_loop(0, _G, init_body, jnp.int32(0))
            mask_v[pl.ds(0, _L)] = jnp.where(lanes == 0, jnp.int32(1), jnp.int32(0))

            def sweep_body(g, ch):
                base = g * _L
                cur = mask_v[pl.ds(base, _L)]
                lv = plsc.load_gather(mask_v, [wl_v[pl.ds(base, _L)]])
                rv = plsc.load_gather(mask_v, [wr_v[pl.ds(base, _L)]])
                new = cur | lv | rv
                mask_v[pl.ds(base, _L)] = new
                return ch | (new ^ cur)

            def w_cond(c):
                return c != 0

            def w_body(c):
                chv = lax.fori_loop(0, _G, sweep_body, jnp.zeros(_L, jnp.int32))
                return jnp.max(chv)

            lax.while_loop(w_cond, w_body, jnp.int32(1))
            pltpu.sync_copy(mask_v, out_hbm)

    return k


_k = _build()


def kernel(thresholds, rules_left, rules_right, binary_ops, left, right):
    wl = _winners(left)
    wr = _winners(right)
    out = _k(wl, wr)
    return out != 0


# speedup vs baseline: 17254.4921x; 24.8807x over previous
import functools

import jax
import jax.numpy as jnp
from jax import lax
from jax.experimental import pallas as pl
from jax.experimental.pallas import tpu as pltpu
from jax.experimental.pallas import tpu_sc as plsc

_N = 10000
_L = 16
_G = _N // _L


def _build():
    mesh = plsc.VectorSubcoreMesh(core_axis_name="c", subcore_axis_name="s")

    @functools.partial(
        pl.kernel,
        mesh=mesh,
        out_type=jax.ShapeDtypeStruct((_N,), jnp.int32),
        compiler_params=pltpu.CompilerParams(needs_layout_passes=False),
        scratch_types=[
            pltpu.VMEM((_N + _L,), jnp.int32),
            pltpu.VMEM((_N,), jnp.int32),
            pltpu.VMEM((_N + _L,), jnp.int32),
            pltpu.VMEM((_N,), jnp.int32),
            pltpu.VMEM((_N,), jnp.int32),
            pltpu.VMEM((_N,), jnp.int32),
            pltpu.VMEM((_N,), jnp.int32),
        ],
    )
    def k(kl_hbm, vl_hbm, kr_hbm, vr_hbm, out_hbm,
          kl_v, vl_v, kr_v, vr_v, wl_v, wr_v, mask_v):
        cid = lax.axis_index("c")
        sid = lax.axis_index("s")

        @pl.when((cid == 0) & (sid == 0))
        def _():
            pltpu.sync_copy(kl_hbm, kl_v.at[pl.ds(0, _N)])
            pltpu.sync_copy(vl_hbm, vl_v)
            pltpu.sync_copy(kr_hbm, kr_v.at[pl.ds(0, _N)])
            pltpu.sync_copy(vr_hbm, vr_v)
            lanes = lax.iota(jnp.int32, _L)
            sentinel = jnp.full((_L,), _N, jnp.int32)
            kl_v[pl.ds(_N, _L)] = sentinel
            kr_v[pl.ds(_N, _L)] = sentinel

            def init_body(g, c):
                base = g * _L
                ids = base + lanes
                wl_v[pl.ds(base, _L)] = ids
                wr_v[pl.ds(base, _L)] = ids
                mask_v[pl.ds(base, _L)] = jnp.zeros(_L, jnp.int32)
                return c

            lax.fori_loop(0, _G, init_body, jnp.int32(0))
            mask_v[pl.ds(0, _L)] = jnp.where(lanes == 0, jnp.int32(1), jnp.int32(0))

            def phase_a(g, c):
                base = g * _L
                k1 = kl_v[pl.ds(base, _L)]
                keep1 = k1 != kl_v[pl.ds(base + 1, _L)]
                plsc.store_scatter(wl_v, [k1], vl_v[pl.ds(base, _L)], mask=keep1)
                k2 = kr_v[pl.ds(base, _L)]
                keep2 = k2 != kr_v[pl.ds(base + 1, _L)]
                plsc.store_scatter(wr_v, [k2], vr_v[pl.ds(base, _L)], mask=keep2)
                return c

            lax.fori_loop(0, _G, phase_a, jnp.int32(0))

            def sweep_body(g, ch):
                base = g * _L
                cur = mask_v[pl.ds(base, _L)]
                lv = plsc.load_gather(mask_v, [wl_v[pl.ds(base, _L)]])
                rv = plsc.load_gather(mask_v, [wr_v[pl.ds(base, _L)]])
                new = cur | lv | rv
                mask_v[pl.ds(base, _L)] = new
                return ch | (new ^ cur)

            def w_cond(c):
                return c != 0

            def w_body(c):
                chv = lax.fori_loop(0, _G, sweep_body, jnp.zeros(_L, jnp.int32))
                return jnp.max(chv)

            lax.while_loop(w_cond, w_body, jnp.int32(1))
            pltpu.sync_copy(mask_v, out_hbm)

    return k


_k = _build()


def kernel(thresholds, rules_left, rules_right, binary_ops, left, right):
    iota = jnp.arange(_N, dtype=jnp.int32)
    kl, vl = lax.sort((left, iota), num_keys=1, is_stable=False)
    kr, vr = lax.sort((right, iota), num_keys=1, is_stable=False)
    out = _k(kl, vl, kr, vr)
    return out != 0
